# Initial kernel scaffold; baseline (speedup 1.0000x reference)
import functools

import jax
import jax.numpy as jnp
from jax import lax
from jax.experimental import pallas as pl
from jax.experimental.pallas import tpu as pltpu
from jax.experimental.pallas import tpu_sc as plsc

VOCAB = 100000
EMB_DIM = 128
KNOW_DIM = 64
OUT_DIM = EMB_DIM + KNOW_DIM
BATCH = 4096
HIST = 50
N = BATCH * HIST
NC, NS = 2, 16
NW = NC * NS
PER_W = N // NW
CHUNK = 128
NCHUNK = PER_W // CHUNK


@functools.partial(
    pl.kernel,
    out_type=jax.ShapeDtypeStruct((NW, NCHUNK, CHUNK, OUT_DIM), jnp.float32),
    mesh=plsc.VectorSubcoreMesh(core_axis_name="c", subcore_axis_name="s"),
    scratch_types=[
        pltpu.VMEM((NCHUNK, CHUNK), jnp.int32),
        pltpu.VMEM((CHUNK, EMB_DIM), jnp.float32),
        pltpu.VMEM((CHUNK, KNOW_DIM), jnp.float32),
        pltpu.SemaphoreType.DMA,
        pltpu.SemaphoreType.DMA,
    ],
)
def _sc_gather(idx_hbm, word_hbm, know_hbm, out_hbm, idx_v, wrows, krows,
               sem_w, sem_k):
    wid = lax.axis_index("s") * NC + lax.axis_index("c")
    pltpu.sync_copy(idx_hbm.at[wid], idx_v)

    def body(j, carry):
        cw = pltpu.async_copy(word_hbm.at[idx_v.at[j]], wrows, sem_w)
        ck = pltpu.async_copy(know_hbm.at[idx_v.at[j]], krows, sem_k)
        cw.wait()
        ck.wait()
        pltpu.sync_copy(wrows, out_hbm.at[wid, j, :, pl.ds(0, EMB_DIM)])
        pltpu.sync_copy(krows, out_hbm.at[wid, j, :, pl.ds(EMB_DIM, KNOW_DIM)])
        return carry

    lax.fori_loop(0, NCHUNK, body, 0)


def kernel(x, word_table, knowledge_table):
    idx = x.reshape(NW, NCHUNK, CHUNK).astype(jnp.int32)
    out = _sc_gather(idx, word_table, knowledge_table)
    return out.reshape(BATCH, HIST, OUT_DIM)


# speedup vs baseline: 3.6217x; 3.6217x over previous
import functools

import jax
import jax.numpy as jnp
from jax import lax
from jax.experimental import pallas as pl
from jax.experimental.pallas import tpu as pltpu
from jax.experimental.pallas import tpu_sc as plsc

VOCAB = 100000
EMB_DIM = 128
KNOW_DIM = 64
OUT_DIM = EMB_DIM + KNOW_DIM
BATCH = 4096
HIST = 50
N = BATCH * HIST
NC, NS = 2, 16
NW = NC * NS
PER_W = N // NW
CHUNK = 128
NCHUNK = PER_W // CHUNK


@functools.partial(
    pl.kernel,
    out_type=jax.ShapeDtypeStruct((NW, NCHUNK, CHUNK, OUT_DIM), jnp.float32),
    mesh=plsc.VectorSubcoreMesh(core_axis_name="c", subcore_axis_name="s"),
    scratch_types=[
        pltpu.VMEM((NCHUNK, CHUNK), jnp.int32),
        pltpu.VMEM((CHUNK, EMB_DIM), jnp.float32),
        pltpu.VMEM((CHUNK, KNOW_DIM), jnp.float32),
        pltpu.SemaphoreType.DMA,
        pltpu.SemaphoreType.DMA,
    ],
    compiler_params=pltpu.CompilerParams(use_tc_tiling_on_sc=False),
)
def _sc_gather(idx_hbm, word_hbm, know_hbm, out_hbm, idx_v, wrows, krows,
               sem_w, sem_k):
    wid = lax.axis_index("s") * NC + lax.axis_index("c")
    pltpu.sync_copy(idx_hbm.at[wid], idx_v)

    def body(j, carry):
        cw = pltpu.async_copy(word_hbm.at[idx_v.at[j]], wrows, sem_w)
        ck = pltpu.async_copy(know_hbm.at[idx_v.at[j]], krows, sem_k)
        cw.wait()
        ck.wait()
        pltpu.sync_copy(wrows, out_hbm.at[wid, j, :, pl.ds(0, EMB_DIM)])
        pltpu.sync_copy(krows, out_hbm.at[wid, j, :, pl.ds(EMB_DIM, KNOW_DIM)])
        return carry

    lax.fori_loop(0, NCHUNK, body, 0)


def kernel(x, word_table, knowledge_table):
    idx = x.reshape(NW, NCHUNK, CHUNK).astype(jnp.int32)
    out = _sc_gather(idx, word_table, knowledge_table)
    return out.reshape(BATCH, HIST, OUT_DIM)


# speedup vs baseline: 3.8071x; 1.0512x over previous
import functools

import jax
import jax.numpy as jnp
from jax import lax
from jax.experimental import pallas as pl
from jax.experimental.pallas import tpu as pltpu
from jax.experimental.pallas import tpu_sc as plsc

VOCAB = 100000
EMB_DIM = 128
KNOW_DIM = 64
OUT_DIM = EMB_DIM + KNOW_DIM
BATCH = 4096
HIST = 50
N = BATCH * HIST
NC, NS = 2, 16
NW = NC * NS
PER_W = N // NW
CHUNK = 128
NCHUNK = PER_W // CHUNK
RING = 5
NGROUP = NCHUNK // RING


@functools.partial(
    pl.kernel,
    out_type=jax.ShapeDtypeStruct((NW, NCHUNK, CHUNK, OUT_DIM), jnp.float32),
    mesh=plsc.VectorSubcoreMesh(core_axis_name="c", subcore_axis_name="s"),
    scratch_types=[
        pltpu.VMEM((NCHUNK, CHUNK), jnp.int32),
        pltpu.VMEM((RING, CHUNK, EMB_DIM), jnp.float32),
        pltpu.VMEM((RING, CHUNK, KNOW_DIM), jnp.float32),
        [pltpu.SemaphoreType.DMA] * RING,
        [pltpu.SemaphoreType.DMA] * RING,
        [pltpu.SemaphoreType.DMA] * RING,
        [pltpu.SemaphoreType.DMA] * RING,
    ],
    compiler_params=pltpu.CompilerParams(use_tc_tiling_on_sc=False),
)
def _sc_gather(idx_hbm, word_hbm, know_hbm, out_hbm, idx_v, wrows, krows,
               sgw, sgk, sww, swk):
    wid = lax.axis_index("s") * NC + lax.axis_index("c")
    pltpu.sync_copy(idx_hbm.at[wid], idx_v)

    def fire_gather(j, b):
        pltpu.async_copy(word_hbm.at[idx_v.at[j]], wrows.at[b], sgw[b])
        pltpu.async_copy(know_hbm.at[idx_v.at[j]], krows.at[b], sgk[b])

    def wait_gather(j, b):
        pltpu.make_async_copy(word_hbm.at[idx_v.at[j]], wrows.at[b],
                              sgw[b]).wait()
        pltpu.make_async_copy(know_hbm.at[idx_v.at[j]], krows.at[b],
                              sgk[b]).wait()

    def fire_write(j, b):
        pltpu.async_copy(wrows.at[b], out_hbm.at[wid, j, :, pl.ds(0, EMB_DIM)],
                         sww[b])
        pltpu.async_copy(krows.at[b],
                         out_hbm.at[wid, j, :, pl.ds(EMB_DIM, KNOW_DIM)],
                         swk[b])

    def wait_write(j, b):
        pltpu.make_async_copy(wrows.at[b],
                              out_hbm.at[wid, j, :, pl.ds(0, EMB_DIM)],
                              sww[b]).wait()
        pltpu.make_async_copy(krows.at[b],
                              out_hbm.at[wid, j, :, pl.ds(EMB_DIM, KNOW_DIM)],
                              swk[b]).wait()

    for b in range(RING):
        fire_gather(b, b)

    @pl.loop(0, NGROUP - 1)
    def _group(g):
        base = g * RING
        for b in range(RING):
            wait_gather(base + b, b)
            fire_write(base + b, b)
        for b in range(RING):
            wait_write(base + b, b)
            fire_gather(base + RING + b, b)

    last = (NGROUP - 1) * RING
    for b in range(RING):
        wait_gather(last + b, b)
        fire_write(last + b, b)
    for b in range(RING):
        wait_write(last + b, b)


def kernel(x, word_table, knowledge_table):
    idx = x.reshape(NW, NCHUNK, CHUNK).astype(jnp.int32)
    out = _sc_gather(idx, word_table, knowledge_table)
    return out.reshape(BATCH, HIST, OUT_DIM)


# speedup vs baseline: 3.8136x; 1.0017x over previous
import functools

import jax
import jax.numpy as jnp
from jax import lax
from jax.experimental import pallas as pl
from jax.experimental.pallas import tpu as pltpu
from jax.experimental.pallas import tpu_sc as plsc

VOCAB = 100000
EMB_DIM = 128
KNOW_DIM = 64
OUT_DIM = EMB_DIM + KNOW_DIM
BATCH = 4096
HIST = 50
N = BATCH * HIST
NC, NS = 2, 16
NW = NC * NS
PER_W = N // NW
CHUNK = 128
NCHUNK = PER_W // CHUNK
RING = 5
NGROUP = NCHUNK // RING


@functools.partial(
    pl.kernel,
    out_type=jax.ShapeDtypeStruct((NW, NCHUNK, CHUNK, OUT_DIM), jnp.float32),
    mesh=plsc.VectorSubcoreMesh(core_axis_name="c", subcore_axis_name="s"),
    scratch_types=[
        pltpu.VMEM((NCHUNK, CHUNK), jnp.int32),
        pltpu.VMEM((RING, CHUNK, EMB_DIM), jnp.float32),
        pltpu.VMEM((RING, CHUNK, KNOW_DIM), jnp.float32),
        [pltpu.SemaphoreType.DMA] * RING,
        [pltpu.SemaphoreType.DMA] * RING,
        [pltpu.SemaphoreType.DMA] * RING,
        [pltpu.SemaphoreType.DMA] * RING,
    ],
    compiler_params=pltpu.CompilerParams(use_tc_tiling_on_sc=False),
)
def _sc_gather(idx_hbm, word_hbm, know_hbm, out_hbm, idx_v, wrows, krows,
               sgw, sgk, sww, swk):
    wid = lax.axis_index("s") * NC + lax.axis_index("c")
    pltpu.sync_copy(idx_hbm.at[wid], idx_v)

    def fire_gather(j, b):
        pltpu.async_copy(word_hbm.at[idx_v.at[j]], wrows.at[b], sgw[b])
        pltpu.async_copy(know_hbm.at[idx_v.at[j]], krows.at[b], sgk[b])

    def wait_gather(j, b):
        pltpu.make_async_copy(word_hbm.at[idx_v.at[j]], wrows.at[b],
                              sgw[b]).wait()
        pltpu.make_async_copy(know_hbm.at[idx_v.at[j]], krows.at[b],
                              sgk[b]).wait()

    def fire_write(j, b):
        pltpu.async_copy(wrows.at[b], out_hbm.at[wid, j, :, pl.ds(0, EMB_DIM)],
                         sww[b])
        pltpu.async_copy(krows.at[b],
                         out_hbm.at[wid, j, :, pl.ds(EMB_DIM, KNOW_DIM)],
                         swk[b])

    def wait_write(j, b):
        pltpu.make_async_copy(wrows.at[b],
                              out_hbm.at[wid, j, :, pl.ds(0, EMB_DIM)],
                              sww[b]).wait()
        pltpu.make_async_copy(krows.at[b],
                              out_hbm.at[wid, j, :, pl.ds(EMB_DIM, KNOW_DIM)],
                              swk[b]).wait()

    for b in range(RING):
        fire_gather(b, b)

    for j in range(NCHUNK):
        k = j - 1
        if k >= 0 and k + RING < NCHUNK:
            wait_write(k, k % RING)
            fire_gather(k + RING, k % RING)
        wait_gather(j, j % RING)
        fire_write(j, j % RING)

    for j in range(NCHUNK - RING, NCHUNK):
        wait_write(j, j % RING)


def kernel(x, word_table, knowledge_table):
    idx = x.reshape(NW, NCHUNK, CHUNK).astype(jnp.int32)
    out = _sc_gather(idx, word_table, knowledge_table)
    return out.reshape(BATCH, HIST, OUT_DIM)
